# Initial kernel scaffold; baseline (speedup 1.0000x reference)
import functools

import jax
import jax.numpy as jnp
from jax.experimental import pallas as pl
from jax.experimental.pallas import tpu as pltpu

DIM = 768
NH = 12
DH = 64
TOPK = 4
P = 16
RT = 256
N = P * RT
HF = 3072

_F32 = jnp.float32
_BF16 = jnp.bfloat16
_HI = jax.lax.Precision.HIGHEST


def _layernorm(x, g, b):
    mu = jnp.mean(x, axis=-1, keepdims=True)
    xc = x - mu
    var = jnp.mean(xc * xc, axis=-1, keepdims=True)
    return xc * jax.lax.rsqrt(var + 1e-5) * g + b


def _qkv_body(x_ref, g_ref, b_ref, w_ref, bq_ref,
              q_ref, k_ref, v_ref, qs_ref, ks_ref):
    xn = _layernorm(x_ref[...], g_ref[...], b_ref[...])
    u = jnp.dot(xn, w_ref[...], preferred_element_type=_F32,
                precision=_HI) + bq_ref[...]
    s = (u > 0.0).astype(_BF16)
    q = s[:, :DIM]
    k = s[:, DIM:2 * DIM]
    v = s[:, 2 * DIM:]
    q_ref[...] = q
    k_ref[...] = k
    v_ref[...] = v
    qs_ref[...] = jnp.sum(q.astype(_F32), axis=0, keepdims=True)
    ks_ref[...] = jnp.sum(k.astype(_F32), axis=0, keepdims=True)


def _run_qkv(x2d, gamma1, beta1, W_qkv, b_qkv):
    return pl.pallas_call(
        _qkv_body,
        grid=(P,),
        in_specs=[
            pl.BlockSpec((RT, DIM), lambda p: (p, 0)),
            pl.BlockSpec((DIM,), lambda p: (0,)),
            pl.BlockSpec((DIM,), lambda p: (0,)),
            pl.BlockSpec((DIM, 3 * DIM), lambda p: (0, 0)),
            pl.BlockSpec((3 * DIM,), lambda p: (0,)),
        ],
        out_specs=[
            pl.BlockSpec((RT, DIM), lambda p: (p, 0)),
            pl.BlockSpec((RT, DIM), lambda p: (p, 0)),
            pl.BlockSpec((RT, DIM), lambda p: (p, 0)),
            pl.BlockSpec((1, DIM), lambda p: (p, 0)),
            pl.BlockSpec((1, DIM), lambda p: (p, 0)),
        ],
        out_shape=[
            jax.ShapeDtypeStruct((N, DIM), _BF16),
            jax.ShapeDtypeStruct((N, DIM), _BF16),
            jax.ShapeDtypeStruct((N, DIM), _BF16),
            jax.ShapeDtypeStruct((P, DIM), _F32),
            jax.ShapeDtypeStruct((P, DIM), _F32),
        ],
    )(x2d, gamma1, beta1, W_qkv, b_qkv)


def _route_body(qs_ref, ks_ref, idx_ref):
    a = jax.lax.dot_general(qs_ref[...], ks_ref[...],
                            (((1,), (1,)), ((), ())),
                            preferred_element_type=_F32, precision=_HI)
    cols = jax.lax.broadcasted_iota(jnp.int32, (P, P), 1)
    out_cols = jax.lax.broadcasted_iota(jnp.int32, (P, 128), 1)
    out = jnp.zeros((P, 128), jnp.int32)
    for t in range(TOPK):
        m = jnp.max(a, axis=-1, keepdims=True)
        j = jnp.min(jnp.where(a == m, cols, P), axis=-1, keepdims=True)
        out = jnp.where(out_cols == t, j, out)
        a = jnp.where(cols == j, -jnp.inf, a)
    idx_ref[...] = out


def _run_route(qs, ks):
    return pl.pallas_call(
        _route_body,
        out_shape=jax.ShapeDtypeStruct((P, 128), jnp.int32),
    )(qs, ks)


def _attn_body(idx_ref, q_ref, k0, k1, k2, k3, v0, v1, v2, v3,
               x_ref, wo_ref, bo_ref, y_ref, kcat, vcat, attn):
    del idx_ref
    for t, (kt, vt) in enumerate(((k0, v0), (k1, v1), (k2, v2), (k3, v3))):
        kcat[t * RT:(t + 1) * RT, :] = kt[...]
        vcat[t * RT:(t + 1) * RT, :] = vt[...]
    q = q_ref[...]
    scale = float(DH) ** -0.5
    for h in range(NH):
        sl = slice(h * DH, (h + 1) * DH)
        s = jax.lax.dot_general(q[:, sl], kcat[:, sl],
                                (((1,), (1,)), ((), ())),
                                preferred_element_type=_F32)
        e = jnp.exp(s * scale)
        den = jnp.sum(e, axis=-1, keepdims=True)
        num = jnp.dot(e.astype(_BF16), vcat[:, sl],
                      preferred_element_type=_F32)
        attn[:, sl] = (num / den).astype(_BF16)
    y_ref[...] = (x_ref[...]
                  + jnp.dot(attn[...], wo_ref[...],
                            preferred_element_type=_F32)
                  + bo_ref[...])


def _run_attn(idx_flat, q, k, v, x2d, Wo_bf16, b_o):
    kv_specs = []
    for t in range(TOPK):
        kv_specs.append(pl.BlockSpec(
            (RT, DIM),
            functools.partial(lambda tt, p, idx: (idx[TOPK * p + tt], 0), t)))
    grid_spec = pltpu.PrefetchScalarGridSpec(
        num_scalar_prefetch=1,
        grid=(P,),
        in_specs=[
            pl.BlockSpec((RT, DIM), lambda p, idx: (p, 0)),
            *kv_specs,
            *kv_specs,
            pl.BlockSpec((RT, DIM), lambda p, idx: (p, 0)),
            pl.BlockSpec((DIM, DIM), lambda p, idx: (0, 0)),
            pl.BlockSpec((DIM,), lambda p, idx: (0,)),
        ],
        out_specs=pl.BlockSpec((RT, DIM), lambda p, idx: (p, 0)),
        scratch_shapes=[
            pltpu.VMEM((TOPK * RT, DIM), _BF16),
            pltpu.VMEM((TOPK * RT, DIM), _BF16),
            pltpu.VMEM((RT, DIM), _BF16),
        ],
    )
    return pl.pallas_call(
        _attn_body,
        grid_spec=grid_spec,
        out_shape=jax.ShapeDtypeStruct((N, DIM), _F32),
    )(idx_flat, q, k, k, k, k, v, v, v, v, x2d, Wo_bf16, b_o)


def _mlp_body(x_ref, g_ref, b_ref, w1_ref, b1_ref, w2_ref, b2_ref, o_ref):
    x = x_ref[...]
    xn = _layernorm(x, g_ref[...], b_ref[...])
    u = jnp.dot(xn, w1_ref[...], preferred_element_type=_F32,
                precision=_HI) + b1_ref[...]
    h = (u > 0.0).astype(_BF16)
    y = jnp.dot(h, w2_ref[...], preferred_element_type=_F32) + b2_ref[...]
    o_ref[...] = x + y


def _run_mlp(y1, gamma2, beta2, W1, b1, W2_bf16, b2):
    return pl.pallas_call(
        _mlp_body,
        grid=(P,),
        in_specs=[
            pl.BlockSpec((RT, DIM), lambda p: (p, 0)),
            pl.BlockSpec((DIM,), lambda p: (0,)),
            pl.BlockSpec((DIM,), lambda p: (0,)),
            pl.BlockSpec((DIM, HF), lambda p: (0, 0)),
            pl.BlockSpec((HF,), lambda p: (0,)),
            pl.BlockSpec((HF, DIM), lambda p: (0, 0)),
            pl.BlockSpec((DIM,), lambda p: (0,)),
        ],
        out_specs=pl.BlockSpec((RT, DIM), lambda p: (p, 0)),
        out_shape=jax.ShapeDtypeStruct((N, DIM), _F32),
    )(y1, gamma2, beta2, W1, b1, W2_bf16, b2)


def kernel(x, gamma1, beta1, W_qkv, b_qkv, W_o, b_o, gamma2, beta2,
           W1, b1, W2, b2):
    x2d = x[0]
    q, k, v, qs, ks = _run_qkv(x2d, gamma1, beta1, W_qkv, b_qkv)
    idx_pad = _run_route(qs, ks)
    idx_flat = idx_pad[:, :TOPK].reshape(-1)
    y1 = _run_attn(idx_flat, q, k, v, x2d, W_o.astype(_BF16), b_o)
    out = _run_mlp(y1, gamma2, beta2, W1, b1, W2.astype(_BF16), b2)
    return out[None]


# speedup vs baseline: 2.7282x; 2.7282x over previous
import functools

import jax
import jax.numpy as jnp
from jax.experimental import pallas as pl
from jax.experimental.pallas import tpu as pltpu

DIM = 768
NH = 12
DH = 64
TOPK = 4
P = 16
RT = 256
N = P * RT
HF = 3072

_F32 = jnp.float32
_BF16 = jnp.bfloat16
_HI = jax.lax.Precision.HIGHEST
_DT_KV = jnp.bfloat16
_DT_E = jnp.bfloat16
_DT_ATTN = jnp.float32
_DT_WO = jnp.float32
_DT_W2 = jnp.bfloat16


def _layernorm(x, g, b):
    mu = jnp.mean(x, axis=-1, keepdims=True)
    xc = x - mu
    var = jnp.mean(xc * xc, axis=-1, keepdims=True)
    return xc * jax.lax.rsqrt(var + 1e-5) * g + b


def _qkv_body(x_ref, g_ref, b_ref, w_ref, bq_ref,
              q_ref, k_ref, v_ref, qs_ref, ks_ref):
    xn = _layernorm(x_ref[...], g_ref[...], b_ref[...])
    u = jnp.dot(xn, w_ref[...], preferred_element_type=_F32) + bq_ref[...]
    s = (u > 0.0).astype(_DT_KV)
    q = s[:, :DIM]
    k = s[:, DIM:2 * DIM]
    v = s[:, 2 * DIM:]
    q_ref[...] = q
    k_ref[...] = k
    v_ref[...] = v
    qs_ref[...] = jnp.sum(q.astype(_F32), axis=0, keepdims=True)[None]
    ks_ref[...] = jnp.sum(k.astype(_F32), axis=0, keepdims=True)[None]


def _run_qkv(x2d, gamma1, beta1, W_qkv, b_qkv):
    return pl.pallas_call(
        _qkv_body,
        grid=(P,),
        in_specs=[
            pl.BlockSpec((RT, DIM), lambda p: (p, 0)),
            pl.BlockSpec((DIM,), lambda p: (0,)),
            pl.BlockSpec((DIM,), lambda p: (0,)),
            pl.BlockSpec((DIM, 3 * DIM), lambda p: (0, 0)),
            pl.BlockSpec((3 * DIM,), lambda p: (0,)),
        ],
        out_specs=[
            pl.BlockSpec((RT, DIM), lambda p: (p, 0)),
            pl.BlockSpec((RT, DIM), lambda p: (p, 0)),
            pl.BlockSpec((RT, DIM), lambda p: (p, 0)),
            pl.BlockSpec((1, 1, DIM), lambda p: (p, 0, 0)),
            pl.BlockSpec((1, 1, DIM), lambda p: (p, 0, 0)),
        ],
        out_shape=[
            jax.ShapeDtypeStruct((N, DIM), _DT_KV),
            jax.ShapeDtypeStruct((N, DIM), _DT_KV),
            jax.ShapeDtypeStruct((N, DIM), _DT_KV),
            jax.ShapeDtypeStruct((P, 1, DIM), _F32),
            jax.ShapeDtypeStruct((P, 1, DIM), _F32),
        ],
    )(x2d, gamma1, beta1, W_qkv, b_qkv)


def _route_body(qs_ref, ks_ref, idx_ref):
    a = jax.lax.dot_general(qs_ref[...] * (1.0 / RT), ks_ref[...] * (1.0 / RT),
                            (((1,), (1,)), ((), ())),
                            preferred_element_type=_F32)
    cols = jax.lax.broadcasted_iota(jnp.int32, (P, P), 1)
    out_cols = jax.lax.broadcasted_iota(jnp.int32, (P, 128), 1)
    out = jnp.zeros((P, 128), jnp.int32)
    for t in range(TOPK):
        m = jnp.max(a, axis=-1, keepdims=True)
        j = jnp.min(jnp.where(a == m, cols, P), axis=-1, keepdims=True)
        out = jnp.where(out_cols == t, j, out)
        a = jnp.where(cols == j, -jnp.inf, a)
    idx_ref[...] = out


def _run_route(qs, ks):
    return pl.pallas_call(
        _route_body,
        out_shape=jax.ShapeDtypeStruct((P, 128), jnp.int32),
    )(qs, ks)


def _attn_body(idx_ref, q_ref, k0, k1, k2, k3, v0, v1, v2, v3,
               x_ref, wo_ref, bo_ref, y_ref, kcat, vcat, attn):
    del idx_ref
    for t, (kt, vt) in enumerate(((k0, v0), (k1, v1), (k2, v2), (k3, v3))):
        kcat[t * RT:(t + 1) * RT, :] = kt[...]
        vcat[t * RT:(t + 1) * RT, :] = vt[...]
    q = q_ref[...]
    scale = float(DH) ** -0.5
    for h in range(NH):
        sl = slice(h * DH, (h + 1) * DH)
        s = jax.lax.dot_general(q[:, sl], kcat[:, sl],
                                (((1,), (1,)), ((), ())),
                                preferred_element_type=_F32)
        ss = s * scale
        m = jnp.max(ss, axis=-1, keepdims=True)
        e = jnp.exp(ss - m)
        den = jnp.sum(e, axis=-1, keepdims=True)
        p_ = e / den
        attn[:, sl] = jnp.dot(p_, vcat[:, sl].astype(_F32),
                              preferred_element_type=_F32)
    y_ref[...] = (x_ref[...]
                  + jnp.dot(attn[...], wo_ref[...],
                            preferred_element_type=_F32)
                  + bo_ref[...])


def _run_attn(idx_flat, q, k, v, x2d, Wo_bf16, b_o):
    kv_specs = []
    for t in range(TOPK):
        kv_specs.append(pl.BlockSpec(
            (RT, DIM),
            functools.partial(lambda tt, p, idx: (idx[TOPK * p + tt], 0), t)))
    grid_spec = pltpu.PrefetchScalarGridSpec(
        num_scalar_prefetch=1,
        grid=(P,),
        in_specs=[
            pl.BlockSpec((RT, DIM), lambda p, idx: (p, 0)),
            *kv_specs,
            *kv_specs,
            pl.BlockSpec((RT, DIM), lambda p, idx: (p, 0)),
            pl.BlockSpec((DIM, DIM), lambda p, idx: (0, 0)),
            pl.BlockSpec((DIM,), lambda p, idx: (0,)),
        ],
        out_specs=pl.BlockSpec((RT, DIM), lambda p, idx: (p, 0)),
        scratch_shapes=[
            pltpu.VMEM((TOPK * RT, DIM), _DT_KV),
            pltpu.VMEM((TOPK * RT, DIM), _DT_KV),
            pltpu.VMEM((RT, DIM), _DT_ATTN),
        ],
    )
    return pl.pallas_call(
        _attn_body,
        grid_spec=grid_spec,
        out_shape=jax.ShapeDtypeStruct((N, DIM), _F32),
    )(idx_flat, q, k, k, k, k, v, v, v, v, x2d, Wo_bf16, b_o)


def _mlp_body(x_ref, g_ref, b_ref, w1_ref, b1_ref, w2_ref, b2_ref, o_ref):
    x = x_ref[...]
    xn = _layernorm(x, g_ref[...], b_ref[...])
    u = jnp.dot(xn, w1_ref[...], preferred_element_type=_F32) + b1_ref[...]
    h = (u > 0.0).astype(_F32)
    y = jnp.dot(h, w2_ref[...], preferred_element_type=_F32) + b2_ref[...]
    o_ref[...] = x + y


def _run_mlp(y1, gamma2, beta2, W1, b1, W2_bf16, b2):
    return pl.pallas_call(
        _mlp_body,
        grid=(P,),
        in_specs=[
            pl.BlockSpec((RT, DIM), lambda p: (p, 0)),
            pl.BlockSpec((DIM,), lambda p: (0,)),
            pl.BlockSpec((DIM,), lambda p: (0,)),
            pl.BlockSpec((DIM, HF), lambda p: (0, 0)),
            pl.BlockSpec((HF,), lambda p: (0,)),
            pl.BlockSpec((HF, DIM), lambda p: (0, 0)),
            pl.BlockSpec((DIM,), lambda p: (0,)),
        ],
        out_specs=pl.BlockSpec((RT, DIM), lambda p: (p, 0)),
        out_shape=jax.ShapeDtypeStruct((N, DIM), _F32),
    )(y1, gamma2, beta2, W1, b1, W2_bf16, b2)


def kernel(x, gamma1, beta1, W_qkv, b_qkv, W_o, b_o, gamma2, beta2,
           W1, b1, W2, b2):
    x2d = x[0]
    q, k, v, qs, ks = _run_qkv(x2d, gamma1, beta1, W_qkv, b_qkv)
    idx_pad = _run_route(qs[:, 0, :], ks[:, 0, :])
    idx_flat = idx_pad[:, :TOPK].reshape(-1)
    y1 = _run_attn(idx_flat, q, k, v, x2d, W_o, b_o)
    out = _run_mlp(y1, gamma2, beta2, W1, b1, W2, b2)
    return out[None]
